# baseline (speedup 1.0000x reference)
import functools

import jax
import jax.numpy as jnp
import numpy as np
from jax import lax
from jax.experimental import pallas as pl
from jax.experimental.pallas import tpu as pltpu
from jax.experimental.pallas import tpu_sc as plsc

B = 8
N = 512 * 512
R = 5
NC = 2
NS = 16
NW = NC * NS
CH = N // NW
SB = 4096
NBLK = N // SB
TOT = B * NBLK
DUMP = R * CH
DSZ = R * CH + 16


def _entry_table():
    key = jax.random.key(42)
    keys = jax.random.split(key, R * B)
    perms = [
        np.asarray(jax.random.permutation(keys[i], N), dtype=np.int64)
        for i in range(R * B)
    ]
    ngroups = B * NW * NBLK
    all_keys = []
    all_entries = []
    j = np.arange(N, dtype=np.int64)
    w = j // CH
    u = j - w * CH
    for r in range(R):
        for b in range(B):
            perm = perms[r * B + b]
            s = perm // SB
            si = perm - s * SB
            grp = (b * NW + w) * NBLK + s
            entry = ((r * CH + u) << 14) | si
            all_keys.append(grp)
            all_entries.append(entry)
    grp = np.concatenate(all_keys)
    entry = np.concatenate(all_entries).astype(np.int32)
    order = np.argsort(grp, kind="stable")
    grp_s = grp[order]
    entry_s = entry[order]
    counts = np.bincount(grp_s, minlength=ngroups)
    P = int(-(-counts.max() // 128) * 128)
    pad = np.int32(DUMP << 14)
    out = np.full((ngroups, P), pad, dtype=np.int32)
    starts = np.zeros(ngroups, dtype=np.int64)
    np.cumsum(counts[:-1], out=starts[1:])
    col = np.arange(grp_s.size, dtype=np.int64) - starts[grp_s]
    out[grp_s, col] = entry_s
    return out.reshape(-1), P


_TABLE, _P = _entry_table()


def _rsqrt16(x):
    i = lax.bitcast_convert_type(x, jnp.int32)
    i = jnp.int32(0x5F3759DF) - lax.shift_right_arithmetic(i, 1)
    y = lax.bitcast_convert_type(i, jnp.float32)
    hx = jnp.float32(0.5) * x
    for _ in range(2):
        y = y * (jnp.float32(1.5) - hx * y * y)
    return y


def _make_body(P):
    def body(pf_hbm, gf_hbm, ent_hbm, out_hbm,
             sp0, sg0, sp1, sg1, eb0, eb1,
             dst_p, dst_g, lin_p, lin_g, acc, sem0, sem1, sem2):
        wid = lax.axis_index("s") * NC + lax.axis_index("c")
        acc[...] = jnp.zeros((16,), jnp.float32)

        def start(t, spb, sgb, ebb, sem):
            b = t // NBLK
            s = t - b * NBLK
            base = b * N + s * SB
            ebase = ((b * NW + wid) * NBLK + s) * P
            pltpu.make_async_copy(
                pf_hbm.at[pl.ds(base, SB)], spb, sem).start()
            pltpu.make_async_copy(
                gf_hbm.at[pl.ds(base, SB)], sgb, sem).start()
            pltpu.make_async_copy(
                ent_hbm.at[pl.ds(ebase, P)], ebb, sem).start()

        def wait(spb, sgb, ebb, sem):
            pltpu.make_async_copy(pf_hbm.at[pl.ds(0, SB)], spb, sem).wait()
            pltpu.make_async_copy(pf_hbm.at[pl.ds(0, SB)], sgb, sem).wait()
            pltpu.make_async_copy(ent_hbm.at[pl.ds(0, P)], ebb, sem).wait()

        def move_block(spb, sgb, ebb):
            @plsc.parallel_loop(0, P // 16, unroll=8)
            def move(k):
                e = ebb[pl.ds(k * 16, 16)]
                si = jnp.bitwise_and(e, jnp.int32(SB - 1))
                di = lax.shift_right_logical(e, 14)
                vp = plsc.load_gather(spb, [si])
                vg = plsc.load_gather(sgb, [si])
                plsc.store_scatter(dst_p, [di], vp)
                plsc.store_scatter(dst_g, [di], vg)

        start(0, sp0, sg0, eb0, sem0)

        def row(b, carry):
            lbase = b * N + wid * CH
            pltpu.make_async_copy(
                pf_hbm.at[pl.ds(lbase, CH)], lin_p, sem2).start()
            pltpu.make_async_copy(
                gf_hbm.at[pl.ds(lbase, CH)], lin_g, sem2).start()

            def duo(g, carry2):
                t = b * NBLK + 2 * g
                start(t + 1, sp1, sg1, eb1, sem1)
                wait(sp0, sg0, eb0, sem0)
                move_block(sp0, sg0, eb0)
                start(jnp.bitwise_and(t + 2, TOT - 1), sp0, sg0, eb0, sem0)
                wait(sp1, sg1, eb1, sem1)
                move_block(sp1, sg1, eb1)
                return carry2

            lax.fori_loop(0, NBLK // 2, duo, 0)

            pltpu.make_async_copy(
                pf_hbm.at[pl.ds(0, CH)], lin_p, sem2).wait()
            pltpu.make_async_copy(
                gf_hbm.at[pl.ds(0, CH)], lin_g, sem2).wait()

            @plsc.parallel_loop(0, CH // 16, unroll=4,
                                carry=jnp.zeros((16,), jnp.float32))
            def math(i, part):
                sl = pl.ds(i * 16, 16)
                lp = lin_p[sl]
                lg = lin_g[sl]
                sp = jnp.zeros((16,), jnp.float32)
                sg = jnp.zeros((16,), jnp.float32)
                prd = []
                grd = []
                for r in range(R):
                    rsl = pl.ds(r * CH + i * 16, 16)
                    pr = lp - dst_p[rsl]
                    gr = lg - dst_g[rsl]
                    prd.append(pr)
                    grd.append(gr)
                    sp = sp + pr * pr
                    sg = sg + gr * gr
                rp = _rsqrt16(sp)
                rg = _rsqrt16(sg)
                ssum = jnp.zeros((16,), jnp.float32)
                for r in range(R):
                    ssum = ssum + jnp.abs(prd[r] * rp - grd[r] * rg)
                return part + ssum

            acc[...] = acc[...] + math
            return carry

        lax.fori_loop(0, B, row, 0)
        wait(sp0, sg0, eb0, sem0)
        pltpu.sync_copy(acc, out_hbm.at[pl.ds(wid * 16, 16)])

    return body


@functools.lru_cache(maxsize=1)
def _rd_loss_sc():
    ents, P = _TABLE, _P
    mesh = plsc.VectorSubcoreMesh(core_axis_name="c", subcore_axis_name="s")
    fn = pl.kernel(
        _make_body(P),
        mesh=mesh,
        compiler_params=pltpu.CompilerParams(needs_layout_passes=False),
        out_type=jax.ShapeDtypeStruct((NW * 16,), jnp.float32),
        scratch_types=[
            pltpu.VMEM((SB,), jnp.float32),
            pltpu.VMEM((SB,), jnp.float32),
            pltpu.VMEM((SB,), jnp.float32),
            pltpu.VMEM((SB,), jnp.float32),
            pltpu.VMEM((P,), jnp.int32),
            pltpu.VMEM((P,), jnp.int32),
            pltpu.VMEM((DSZ,), jnp.float32),
            pltpu.VMEM((DSZ,), jnp.float32),
            pltpu.VMEM((CH,), jnp.float32),
            pltpu.VMEM((CH,), jnp.float32),
            pltpu.VMEM((16,), jnp.float32),
            pltpu.SemaphoreType.DMA,
            pltpu.SemaphoreType.DMA,
            pltpu.SemaphoreType.DMA,
        ],
    )
    return fn, ents


def kernel(pred, gt):
    pf = pred.reshape(B * N)
    gf = gt.reshape(B * N)
    fn, ents = _rd_loss_sc()
    partials = fn(pf, gf, ents)
    return jnp.sum(partials) / np.float32(B * N * R)


# speedup vs baseline: 1.1272x; 1.1272x over previous
import functools

import jax
import jax.numpy as jnp
import numpy as np
from jax import lax
from jax.experimental import pallas as pl
from jax.experimental.pallas import tpu as pltpu
from jax.experimental.pallas import tpu_sc as plsc

B = 8
N = 512 * 512
R = 5
NC = 2
NS = 16
NW = NC * NS
CH = N // NW
HC = 2048
NHC = CH // HC
SB = 8192
NBLK = N // SB
TOT = B * NBLK
DUMP = R * CH
DSZ = R * CH + 16


def _entry_table():
    key = jax.random.key(42)
    keys = jax.random.split(key, R * B)
    perms = [
        np.asarray(jax.random.permutation(keys[i], N), dtype=np.int64)
        for i in range(R * B)
    ]
    ngroups = B * NW * NBLK
    all_keys = []
    all_entries = []
    j = np.arange(N, dtype=np.int64)
    w = j // CH
    u = j - w * CH
    for r in range(R):
        for b in range(B):
            perm = perms[r * B + b]
            s = perm // SB
            si = perm - s * SB
            grp = (b * NW + w) * NBLK + s
            entry = ((r * CH + u) << 14) | si
            all_keys.append(grp)
            all_entries.append(entry)
    grp = np.concatenate(all_keys)
    entry = np.concatenate(all_entries).astype(np.int32)
    order = np.argsort(grp, kind="stable")
    grp_s = grp[order]
    entry_s = entry[order]
    counts = np.bincount(grp_s, minlength=ngroups)
    P = int(-(-counts.max() // 128) * 128)
    pad = np.int32(DUMP << 14)
    out = np.full((ngroups, P), pad, dtype=np.int32)
    starts = np.zeros(ngroups, dtype=np.int64)
    np.cumsum(counts[:-1], out=starts[1:])
    col = np.arange(grp_s.size, dtype=np.int64) - starts[grp_s]
    out[grp_s, col] = entry_s
    return out.reshape(-1), P


_TABLE, _P = _entry_table()


def _rsqrt16(x):
    i = lax.bitcast_convert_type(x, jnp.int32)
    i = jnp.int32(0x5F3759DF) - lax.shift_right_arithmetic(i, 1)
    y = lax.bitcast_convert_type(i, jnp.float32)
    hx = jnp.float32(0.5) * x
    for _ in range(2):
        y = y * (jnp.float32(1.5) - hx * y * y)
    return y


def _make_body(P):
    def body(pf_hbm, gf_hbm, ent_hbm, out_hbm,
             sp0, sg0, sp1, sg1, eb0, eb1,
             dst_p, dst_g, lp0, lg0, lp1, lg1, acc,
             sem0, sem1, sem2, sem3):
        wid = lax.axis_index("s") * NC + lax.axis_index("c")
        acc[...] = jnp.zeros((16,), jnp.float32)

        def start(t, spb, sgb, ebb, sem):
            b = t // NBLK
            s = t - b * NBLK
            base = b * N + s * SB
            ebase = ((b * NW + wid) * NBLK + s) * P
            pltpu.make_async_copy(
                pf_hbm.at[pl.ds(base, SB)], spb, sem).start()
            pltpu.make_async_copy(
                gf_hbm.at[pl.ds(base, SB)], sgb, sem).start()
            pltpu.make_async_copy(
                ent_hbm.at[pl.ds(ebase, P)], ebb, sem).start()

        def wait(spb, sgb, ebb, sem):
            pltpu.make_async_copy(pf_hbm.at[pl.ds(0, SB)], spb, sem).wait()
            pltpu.make_async_copy(pf_hbm.at[pl.ds(0, SB)], sgb, sem).wait()
            pltpu.make_async_copy(ent_hbm.at[pl.ds(0, P)], ebb, sem).wait()

        def move_block(spb, sgb, ebb):
            @plsc.parallel_loop(0, P // 16, unroll=8)
            def move(k):
                e = ebb[pl.ds(k * 16, 16)]
                si = jnp.bitwise_and(e, jnp.int32(SB - 1))
                di = lax.shift_right_logical(e, 14)
                vp = plsc.load_gather(spb, [si])
                vg = plsc.load_gather(sgb, [si])
                plsc.store_scatter(dst_p, [di], vp)
                plsc.store_scatter(dst_g, [di], vg)

        lin_bufs = [(lp0, lg0, sem2), (lp1, lg1, sem3)]

        def lin_start(b, h):
            lpb, lgb, sem = lin_bufs[h % 2]
            lbase = b * N + wid * CH + h * HC
            pltpu.make_async_copy(
                pf_hbm.at[pl.ds(lbase, HC)], lpb, sem).start()
            pltpu.make_async_copy(
                gf_hbm.at[pl.ds(lbase, HC)], lgb, sem).start()

        def lin_wait(h):
            lpb, lgb, sem = lin_bufs[h % 2]
            pltpu.make_async_copy(pf_hbm.at[pl.ds(0, HC)], lpb, sem).wait()
            pltpu.make_async_copy(gf_hbm.at[pl.ds(0, HC)], lgb, sem).wait()

        start(0, sp0, sg0, eb0, sem0)

        def row(b, carry):
            lin_start(b, 0)

            def duo(g, carry2):
                t = b * NBLK + 2 * g
                start(t + 1, sp1, sg1, eb1, sem1)
                wait(sp0, sg0, eb0, sem0)
                move_block(sp0, sg0, eb0)
                start(jnp.bitwise_and(t + 2, TOT - 1), sp0, sg0, eb0, sem0)
                wait(sp1, sg1, eb1, sem1)
                move_block(sp1, sg1, eb1)
                return carry2

            lax.fori_loop(0, NBLK // 2, duo, 0)

            for h in range(NHC):
                if h + 1 < NHC:
                    lin_start(b, h + 1)
                lin_wait(h)
                lpb, lgb, _ = lin_bufs[h % 2]

                @plsc.parallel_loop(0, HC // 16, unroll=4,
                                    carry=jnp.zeros((16,), jnp.float32))
                def math(i, part):
                    sl = pl.ds(i * 16, 16)
                    lp = lpb[sl]
                    lg = lgb[sl]
                    sp = jnp.zeros((16,), jnp.float32)
                    sg = jnp.zeros((16,), jnp.float32)
                    prd = []
                    grd = []
                    for r in range(R):
                        rsl = pl.ds(r * CH + h * HC + i * 16, 16)
                        pr = lp - dst_p[rsl]
                        gr = lg - dst_g[rsl]
                        prd.append(pr)
                        grd.append(gr)
                        sp = sp + pr * pr
                        sg = sg + gr * gr
                    rp = _rsqrt16(sp)
                    rg = _rsqrt16(sg)
                    ssum = jnp.zeros((16,), jnp.float32)
                    for r in range(R):
                        ssum = ssum + jnp.abs(prd[r] * rp - grd[r] * rg)
                    return part + ssum

                acc[...] = acc[...] + math
            return carry

        lax.fori_loop(0, B, row, 0)
        wait(sp0, sg0, eb0, sem0)
        pltpu.sync_copy(acc, out_hbm.at[pl.ds(wid * 16, 16)])

    return body


@functools.lru_cache(maxsize=1)
def _rd_loss_sc():
    ents, P = _TABLE, _P
    mesh = plsc.VectorSubcoreMesh(core_axis_name="c", subcore_axis_name="s")
    fn = pl.kernel(
        _make_body(P),
        mesh=mesh,
        compiler_params=pltpu.CompilerParams(needs_layout_passes=False),
        out_type=jax.ShapeDtypeStruct((NW * 16,), jnp.float32),
        scratch_types=[
            pltpu.VMEM((SB,), jnp.float32),
            pltpu.VMEM((SB,), jnp.float32),
            pltpu.VMEM((SB,), jnp.float32),
            pltpu.VMEM((SB,), jnp.float32),
            pltpu.VMEM((P,), jnp.int32),
            pltpu.VMEM((P,), jnp.int32),
            pltpu.VMEM((DSZ,), jnp.float32),
            pltpu.VMEM((DSZ,), jnp.float32),
            pltpu.VMEM((HC,), jnp.float32),
            pltpu.VMEM((HC,), jnp.float32),
            pltpu.VMEM((HC,), jnp.float32),
            pltpu.VMEM((HC,), jnp.float32),
            pltpu.VMEM((16,), jnp.float32),
            pltpu.SemaphoreType.DMA,
            pltpu.SemaphoreType.DMA,
            pltpu.SemaphoreType.DMA,
            pltpu.SemaphoreType.DMA,
        ],
    )
    return fn, ents


def kernel(pred, gt):
    pf = pred.reshape(B * N)
    gf = gt.reshape(B * N)
    fn, ents = _rd_loss_sc()
    partials = fn(pf, gf, ents)
    return jnp.sum(partials) / np.float32(B * N * R)
